# baseline (speedup 1.0000x reference)
import jax
import jax.numpy as jnp
from jax import lax
from jax.experimental import pallas as pl
from jax.experimental.pallas import tpu as pltpu
from jax.experimental.pallas import tpu_sc as plsc

B = 8
N = 2048
M = 2048

L = 16
NC = 1
NS = 16
NBLK = 8
NWORK = NC * NS
NSC = 512
NW = NSC // NWORK
NCHUNK = M // L
NCOMP = 4

TILE = 1024
NT = N // TILE
NSC_TILES = NSC // TILE


def _round_bf16(v):
    u = plsc.bitcast(v, jnp.int32)
    lsb = jnp.bitwise_and(lax.shift_right_logical(u, 16), jnp.int32(1))
    r = u + jnp.int32(0x7FFF) + lsb
    r = jnp.bitwise_and(r, jnp.int32(-65536))
    return plsc.bitcast(r, jnp.float32)


def _sc_body(a_ref, b_ref, d1_ref, avx, avy, avz, avs, bvx, bvy, bvz, bvs,
             d1v, t16):
    c = lax.axis_index("c")
    s = lax.axis_index("s")
    wid = c * NS + s
    n0 = wid * NW

    for comp, dst in ((0, avx), (1, avy), (2, avz), (3, avs)):
        pltpu.sync_copy(a_ref.at[pl.ds(comp * NSC + n0, NW)], dst)
    for comp, dst in ((0, bvx), (1, bvy), (2, bvz), (3, bvs)):
        pltpu.sync_copy(b_ref.at[pl.ds(comp * M, M)], dst)

    def round_a(j, carry):
        sl = pl.ds(j * L, L)
        for ref in (avx, avy, avz):
            ref[sl] = _round_bf16(ref[sl])
        return carry

    def round_b(j, carry):
        sl = pl.ds(j * L, L)
        for ref in (bvx, bvy, bvz):
            ref[sl] = _round_bf16(ref[sl])
        return carry

    lax.fori_loop(0, NW // L, round_a, 0)
    lax.fori_loop(0, NCHUNK, round_b, 0)

    lanes = lax.iota(jnp.int32, L)
    row_idx = lanes * L

    def nb_body(nb, carry):
        base = nb * L
        axv = avx[pl.ds(base, L)]
        ayv = avy[pl.ds(base, L)]
        azv = avz[pl.ds(base, L)]
        asv = avs[pl.ds(base, L)]

        for half in range(L // NBLK):
            ax = [axv[half * NBLK + i] for i in range(NBLK)]
            ay = [ayv[half * NBLK + i] for i in range(NBLK)]
            az = [azv[half * NBLK + i] for i in range(NBLK)]
            asq = [asv[half * NBLK + i] for i in range(NBLK)]

            def m_body(j, accs):
                sl = pl.ds(j * L, L)
                bx = bvx[sl]
                by = bvy[sl]
                bz = bvz[sl]
                bsq = bvs[sl]
                new_accs = []
                for i in range(NBLK):
                    t = ax[i] * bx + ay[i] * by + az[i] * bz
                    d = (asq[i] + bsq) - (t + t)
                    new_accs.append(jnp.minimum(accs[i], d))
                return tuple(new_accs)

            inf = jnp.full((L,), jnp.inf, jnp.float32)
            accs = lax.fori_loop(0, NCHUNK, m_body, (inf,) * NBLK)
            for i in range(NBLK):
                t16[pl.ds((half * NBLK + i) * L, L)] = accs[i]

        colmin = plsc.load_gather(t16, [row_idx])
        for k in range(1, L):
            colmin = jnp.minimum(colmin,
                                 plsc.load_gather(t16, [row_idx + k]))
        d1v[pl.ds(base, L)] = jnp.maximum(colmin, jnp.float32(0.0))
        return carry

    lax.fori_loop(0, NW // L, nb_body, 0)

    pltpu.sync_copy(d1v, d1_ref.at[pl.ds(n0, NW)])


def _make_sc():
    mesh = plsc.VectorSubcoreMesh(core_axis_name="c", subcore_axis_name="s",
                                  num_cores=NC, num_subcores=NS)
    return pl.kernel(
        _sc_body,
        out_type=jax.ShapeDtypeStruct((NSC,), jnp.float32),
        mesh=mesh,
        compiler_params=pltpu.CompilerParams(needs_layout_passes=False),
        scratch_types=[
            pltpu.VMEM((NW,), jnp.float32),
            pltpu.VMEM((NW,), jnp.float32),
            pltpu.VMEM((NW,), jnp.float32),
            pltpu.VMEM((NW,), jnp.float32),
            pltpu.VMEM((M,), jnp.float32),
            pltpu.VMEM((M,), jnp.float32),
            pltpu.VMEM((M,), jnp.float32),
            pltpu.VMEM((M,), jnp.float32),
            pltpu.VMEM((NW,), jnp.float32),
            pltpu.VMEM((L * L,), jnp.float32),
        ],
    )


def _tc_body(a_ref, bt_ref, d1_ref, d2_ref):
    b = pl.program_id(0)
    nt = pl.program_id(1)
    a = a_ref[0]
    bt = bt_ref[0]
    STRIP = 512
    m1 = None
    m2s = []
    for s in range(M // STRIP):
        bs = lax.slice(bt, (s * STRIP, 0), ((s + 1) * STRIP, 9))
        r = lax.dot_general(a, bs, (((1,), (1,)), ((), ())),
                            preferred_element_type=jnp.float32)
        p1 = jnp.max(r, axis=1)
        m1 = p1 if m1 is None else jnp.maximum(m1, p1)
        m2s.append(jnp.max(r, axis=0))
    m2 = jnp.concatenate(m2s)

    @pl.when(jnp.logical_or(b > 0, nt >= NSC_TILES))
    def _():
        d1_ref[0, 0] = jnp.maximum(-m1, 0.0)

    @pl.when(nt == 0)
    def _():
        d2_ref[0, 0] = m2

    @pl.when(nt > 0)
    def _():
        d2_ref[0, 0] = jnp.maximum(d2_ref[0, 0], m2)

    @pl.when(nt == NT - 1)
    def _():
        d2_ref[0, 0] = jnp.maximum(-d2_ref[0, 0], 0.0)


def _tc_nnd(a9, b9):
    return pl.pallas_call(
        _tc_body,
        grid=(B, NT),
        in_specs=[
            pl.BlockSpec((1, TILE, 9), lambda b, nt: (b, nt, 0)),
            pl.BlockSpec((1, M, 9), lambda b, nt: (b, 0, 0)),
        ],
        out_specs=[
            pl.BlockSpec((1, 1, TILE), lambda b, nt: (b * NT + nt, 0, 0)),
            pl.BlockSpec((1, 1, M), lambda b, nt: (b, 0, 0)),
        ],
        out_shape=[
            jax.ShapeDtypeStruct((B * NT, 1, TILE), jnp.float32),
            jax.ShapeDtypeStruct((B, 1, M), jnp.float32),
        ],
        compiler_params=pltpu.CompilerParams(
            dimension_semantics=("parallel", "arbitrary")),
    )(a9, b9)


def _split3_bf16(x):
    mask = jnp.uint32(0xFFFF0000)
    h = lax.bitcast_convert_type(
        jnp.bitwise_and(lax.bitcast_convert_type(x, jnp.uint32), mask),
        jnp.float32)
    r1 = x - h
    m = lax.bitcast_convert_type(
        jnp.bitwise_and(lax.bitcast_convert_type(r1, jnp.uint32), mask),
        jnp.float32)
    r2 = r1 - m
    return (h.astype(jnp.bfloat16), m.astype(jnp.bfloat16),
            r2.astype(jnp.bfloat16))


def _pack_sc(p, npts):
    comps = jnp.moveaxis(p, -1, 0)
    sq = jnp.sum(p * p, axis=-1)[None, :]
    return jnp.concatenate([comps, sq], axis=0).reshape(NCOMP * npts)


def kernel(input1, input2):
    d1_sc = _make_sc()(_pack_sc(input1[0, :NSC], NSC),
                       _pack_sc(input2[0], M))

    a2 = input1.astype(jnp.bfloat16) * jnp.bfloat16(2)
    sq1 = jnp.sum(input1 * input1, axis=-1)
    h1, m1, l1 = _split3_bf16(sq1)
    ones = jnp.ones((B, N, 3), jnp.bfloat16)
    a9 = jnp.concatenate(
        [a2, jnp.stack([h1, m1, l1], axis=-1), ones], axis=-1)

    b_bf = input2.astype(jnp.bfloat16)
    sq2 = jnp.sum(input2 * input2, axis=-1)
    h2, m2, l2 = _split3_bf16(sq2)
    negones = -jnp.ones((B, M, 3), jnp.bfloat16)
    b9 = jnp.concatenate(
        [b_bf, negones, -jnp.stack([h2, m2, l2], axis=-1)], axis=-1)

    d1_tc, d2 = _tc_nnd(a9, b9)
    d1_tc = d1_tc.reshape(B, N)

    dist1 = jnp.concatenate(
        [jnp.concatenate([d1_sc, d1_tc[0, NSC:]])[None, :], d1_tc[1:]], axis=0)
    return dist1, d2.reshape(B, M)


# speedup vs baseline: 1.0309x; 1.0309x over previous
import jax
import jax.numpy as jnp
from jax import lax
from jax.experimental import pallas as pl
from jax.experimental.pallas import tpu as pltpu
from jax.experimental.pallas import tpu_sc as plsc

B = 8
N = 2048
M = 2048

L = 16
NC = 1
NS = 16
NBLK = 8
NWORK = NC * NS
NSC = 512
NW = NSC // NWORK
NCHUNK = M // L
NCOMP = 4

TILE = 1024
NT = N // TILE
NSC_TILES = NSC // TILE


def _round_bf16(v):
    u = plsc.bitcast(v, jnp.int32)
    lsb = jnp.bitwise_and(lax.shift_right_logical(u, 16), jnp.int32(1))
    r = u + jnp.int32(0x7FFF) + lsb
    r = jnp.bitwise_and(r, jnp.int32(-65536))
    return plsc.bitcast(r, jnp.float32)


def _sc_body(a_ref, b_ref, d1_ref, avx, avy, avz, avs, bvx, bvy, bvz, bvs,
             d1v, t16):
    c = lax.axis_index("c")
    s = lax.axis_index("s")
    wid = c * NS + s
    n0 = wid * NW

    for comp, dst in ((0, avx), (1, avy), (2, avz), (3, avs)):
        pltpu.sync_copy(a_ref.at[pl.ds(comp * NSC + n0, NW)], dst)
    for comp, dst in ((0, bvx), (1, bvy), (2, bvz), (3, bvs)):
        pltpu.sync_copy(b_ref.at[pl.ds(comp * M, M)], dst)

    def round_a(j, carry):
        sl = pl.ds(j * L, L)
        for ref in (avx, avy, avz):
            ref[sl] = _round_bf16(ref[sl])
        return carry

    def round_b(j, carry):
        sl = pl.ds(j * L, L)
        for ref in (bvx, bvy, bvz):
            ref[sl] = _round_bf16(ref[sl])
        return carry

    lax.fori_loop(0, NW // L, round_a, 0)
    lax.fori_loop(0, NCHUNK, round_b, 0)

    lanes = lax.iota(jnp.int32, L)
    row_idx = lanes * L

    def nb_body(nb, carry):
        base = nb * L
        axv = avx[pl.ds(base, L)]
        ayv = avy[pl.ds(base, L)]
        azv = avz[pl.ds(base, L)]
        asv = avs[pl.ds(base, L)]

        for half in range(L // NBLK):
            ax = [axv[half * NBLK + i] for i in range(NBLK)]
            ay = [ayv[half * NBLK + i] for i in range(NBLK)]
            az = [azv[half * NBLK + i] for i in range(NBLK)]
            asq = [asv[half * NBLK + i] for i in range(NBLK)]

            def m_body(j, accs):
                sl = pl.ds(j * L, L)
                bx = bvx[sl]
                by = bvy[sl]
                bz = bvz[sl]
                bsq = bvs[sl]
                new_accs = []
                for i in range(NBLK):
                    t = ax[i] * bx + ay[i] * by + az[i] * bz
                    d = (asq[i] + bsq) - (t + t)
                    new_accs.append(jnp.minimum(accs[i], d))
                return tuple(new_accs)

            inf = jnp.full((L,), jnp.inf, jnp.float32)
            accs = lax.fori_loop(0, NCHUNK, m_body, (inf,) * NBLK)
            for i in range(NBLK):
                t16[pl.ds((half * NBLK + i) * L, L)] = accs[i]

        colmin = plsc.load_gather(t16, [row_idx])
        for k in range(1, L):
            colmin = jnp.minimum(colmin,
                                 plsc.load_gather(t16, [row_idx + k]))
        d1v[pl.ds(base, L)] = jnp.maximum(colmin, jnp.float32(0.0))
        return carry

    lax.fori_loop(0, NW // L, nb_body, 0)

    pltpu.sync_copy(d1v, d1_ref.at[pl.ds(n0, NW)])


def _make_sc():
    mesh = plsc.VectorSubcoreMesh(core_axis_name="c", subcore_axis_name="s",
                                  num_cores=NC, num_subcores=NS)
    return pl.kernel(
        _sc_body,
        out_type=jax.ShapeDtypeStruct((NSC,), jnp.float32),
        mesh=mesh,
        compiler_params=pltpu.CompilerParams(needs_layout_passes=False),
        scratch_types=[
            pltpu.VMEM((NW,), jnp.float32),
            pltpu.VMEM((NW,), jnp.float32),
            pltpu.VMEM((NW,), jnp.float32),
            pltpu.VMEM((NW,), jnp.float32),
            pltpu.VMEM((M,), jnp.float32),
            pltpu.VMEM((M,), jnp.float32),
            pltpu.VMEM((M,), jnp.float32),
            pltpu.VMEM((M,), jnp.float32),
            pltpu.VMEM((NW,), jnp.float32),
            pltpu.VMEM((L * L,), jnp.float32),
        ],
    )


def _tc_body(a_ref, bt_ref, d1_ref, d2_ref):
    b = pl.program_id(0)
    nt = pl.program_id(1)
    a = a_ref[0]
    bt = bt_ref[0]
    STRIP = 512
    m1 = None
    m2s = []
    for s in range(M // STRIP):
        bs = lax.slice(bt, (0, s * STRIP), (9, (s + 1) * STRIP))
        r = lax.dot_general(a, bs, (((1,), (0,)), ((), ())),
                            preferred_element_type=jnp.float32)
        p1 = jnp.max(r, axis=1)
        m1 = p1 if m1 is None else jnp.maximum(m1, p1)
        m2s.append(jnp.max(r, axis=0))
    m2 = jnp.concatenate(m2s)

    @pl.when(jnp.logical_or(b > 0, nt >= NSC_TILES))
    def _():
        d1_ref[0, 0] = jnp.maximum(-m1, 0.0)

    @pl.when(nt == 0)
    def _():
        d2_ref[0, 0] = m2

    @pl.when(nt > 0)
    def _():
        d2_ref[0, 0] = jnp.maximum(d2_ref[0, 0], m2)

    @pl.when(nt == NT - 1)
    def _():
        d2_ref[0, 0] = jnp.maximum(-d2_ref[0, 0], 0.0)


def _tc_nnd(a9, b9):
    return pl.pallas_call(
        _tc_body,
        grid=(B, NT),
        in_specs=[
            pl.BlockSpec((1, TILE, 9), lambda b, nt: (b, nt, 0)),
            pl.BlockSpec((1, 9, M), lambda b, nt: (b, 0, 0)),
        ],
        out_specs=[
            pl.BlockSpec((1, 1, TILE), lambda b, nt: (b * NT + nt, 0, 0)),
            pl.BlockSpec((1, 1, M), lambda b, nt: (b, 0, 0)),
        ],
        out_shape=[
            jax.ShapeDtypeStruct((B * NT, 1, TILE), jnp.float32),
            jax.ShapeDtypeStruct((B, 1, M), jnp.float32),
        ],
        compiler_params=pltpu.CompilerParams(
            dimension_semantics=("parallel", "arbitrary")),
    )(a9, b9)


def _split3_bf16(x):
    mask = jnp.uint32(0xFFFF0000)
    h = lax.bitcast_convert_type(
        jnp.bitwise_and(lax.bitcast_convert_type(x, jnp.uint32), mask),
        jnp.float32)
    r1 = x - h
    m = lax.bitcast_convert_type(
        jnp.bitwise_and(lax.bitcast_convert_type(r1, jnp.uint32), mask),
        jnp.float32)
    r2 = r1 - m
    return (h.astype(jnp.bfloat16), m.astype(jnp.bfloat16),
            r2.astype(jnp.bfloat16))


def _pack_sc(p, npts):
    comps = jnp.moveaxis(p, -1, 0)
    sq = jnp.sum(p * p, axis=-1)[None, :]
    return jnp.concatenate([comps, sq], axis=0).reshape(NCOMP * npts)


def kernel(input1, input2):
    d1_sc = _make_sc()(_pack_sc(input1[0, :NSC], NSC),
                       _pack_sc(input2[0], M))

    a2 = input1.astype(jnp.bfloat16) * jnp.bfloat16(2)
    sq1 = jnp.sum(input1 * input1, axis=-1)
    h1, m1, l1 = _split3_bf16(sq1)
    ones = jnp.ones((B, N, 3), jnp.bfloat16)
    a9 = jnp.concatenate(
        [a2, jnp.stack([h1, m1, l1], axis=-1), ones], axis=-1)

    bt = jnp.transpose(input2, (0, 2, 1))
    b_bf = bt.astype(jnp.bfloat16)
    sq2 = jnp.sum(input2 * input2, axis=-1)
    h2, m2, l2 = _split3_bf16(sq2)
    negones = -jnp.ones((B, 3, M), jnp.bfloat16)
    b9 = jnp.concatenate(
        [b_bf, negones, -jnp.stack([h2, m2, l2], axis=1)], axis=1)

    d1_tc, d2 = _tc_nnd(a9, b9)
    d1_tc = d1_tc.reshape(B, N)

    dist1 = jnp.concatenate(
        [jnp.concatenate([d1_sc, d1_tc[0, NSC:]])[None, :], d1_tc[1:]], axis=0)
    return dist1, d2.reshape(B, M)
